# Initial kernel scaffold; baseline (speedup 1.0000x reference)
import jax
import jax.numpy as jnp
from jax.experimental import pallas as pl


def kernel(x, edge_index0, edge_index1, edge_index2, W1, att_src1, att_dst1, b1, W2, att_src2, att_dst2, b2, W3, att_src3, att_dst3, b3, hop_att0, hop_bias0, hop_att1, hop_bias1, bn_weight, bn_bias):
    raise NotImplementedError("write your pallas kernel here")


# speedup vs baseline: 13.4834x; 13.4834x over previous
import functools

import numpy as np
import jax
import jax.numpy as jnp
from jax import lax
from jax.experimental import pallas as pl
from jax.experimental.pallas import tpu as pltpu
from jax.experimental.pallas import tpu_sc as plsc

_N0, _N1, _N2, _N3 = 10000, 4000, 1000, 250
_E0, _E1, _E2 = 128000, 32000, 8000
_HID, _HEADS, _OUT = 32, 4, 128
_DW0 = float(np.log(1.0 / 1.0 + 1.0 + 1e-09))
_NC, _NS = 2, 16
_NW = _NC * _NS

_F32 = jnp.float32
_HIGH = lax.Precision.HIGHEST

_SEL4 = np.repeat(np.eye(_HEADS, dtype=np.float32), _HID, axis=0)
_SELT4 = np.ascontiguousarray(_SEL4.T)


def _tcA_body(x_ref, w_ref, asf_ref, adf_ref, sel_ref, h_ref, asrc_ref,
              adst_ref):
    h = jnp.dot(x_ref[...], w_ref[...], preferred_element_type=_F32,
                precision=_HIGH)
    h_ref[...] = h
    asrc_ref[...] = jnp.dot(h * asf_ref[...], sel_ref[...],
                            preferred_element_type=_F32, precision=_HIGH)
    adst_ref[...] = jnp.dot(h * adf_ref[...], sel_ref[...],
                            preferred_element_type=_F32, precision=_HIGH)


def _tcA(x, W1, asf, adf):
    blk = 2000
    n = x.shape[0]
    return pl.pallas_call(
        _tcA_body,
        grid=(n // blk,),
        in_specs=[
            pl.BlockSpec((blk, 128), lambda i: (i, 0)),
            pl.BlockSpec((128, 128), lambda i: (0, 0)),
            pl.BlockSpec((1, 128), lambda i: (0, 0)),
            pl.BlockSpec((1, 128), lambda i: (0, 0)),
            pl.BlockSpec((128, _HEADS), lambda i: (0, 0)),
        ],
        out_specs=[
            pl.BlockSpec((blk, 128), lambda i: (i, 0)),
            pl.BlockSpec((blk, _HEADS), lambda i: (i, 0)),
            pl.BlockSpec((blk, _HEADS), lambda i: (i, 0)),
        ],
        out_shape=[
            jax.ShapeDtypeStruct((n, 128), _F32),
            jax.ShapeDtypeStruct((n, _HEADS), _F32),
            jax.ShapeDtypeStruct((n, _HEADS), _F32),
        ],
    )(x, W1, asf, adf, jnp.asarray(_SEL4))


def _elu(x):
    return jnp.where(x > 0, x, jnp.exp(x) - 1.0)


def _tcB_body(p0_ref, p1_ref, d0_ref, d1_ref, selt_ref, b1_ref, hat0_ref,
              hb0_ref, bnw_ref, bnb_ref, w2_ref, asf_ref, adf_ref, sel_ref,
              h_ref, asrc_ref, adst_ref, zscale_ref, zsum_ref):
    num = p0_ref[...] + p1_ref[...]
    den = (d0_ref[...] + d1_ref[...])[:, :_HEADS]
    denb = jnp.dot(den, selt_ref[...], preferred_element_type=_F32,
                   precision=_HIGH)
    h1 = num / (denb + 1e-16) + b1_ref[...]
    ga = jnp.dot(_elu(h1), hat0_ref[...], preferred_element_type=_F32,
                 precision=_HIGH) + hb0_ref[...]
    z = h1 * ga
    zscale_ref[...] = z * _DW0
    zsum_ref[...] = z
    bn = (h1 / np.float32(np.sqrt(1.0 + 1e-05))) * bnw_ref[...] + bnb_ref[...]
    h1p = _elu(bn)
    h2 = jnp.dot(h1p, w2_ref[...], preferred_element_type=_F32,
                 precision=_HIGH)
    h_ref[...] = h2
    asrc_ref[...] = jnp.dot(h2 * asf_ref[...], sel_ref[...],
                            preferred_element_type=_F32, precision=_HIGH)
    adst_ref[...] = jnp.dot(h2 * adf_ref[...], sel_ref[...],
                            preferred_element_type=_F32, precision=_HIGH)


def _tcB(p0, p1, d0, d1, b1, hat0T, hb0, bnw, bnb, W2, asf2, adf2):
    blk = 2000
    n = p0.shape[0]
    full = lambda shape: pl.BlockSpec(shape, lambda i: tuple(0 for _ in shape))
    row = lambda w: pl.BlockSpec((blk, w), lambda i: (i, 0))
    return pl.pallas_call(
        _tcB_body,
        grid=(n // blk,),
        in_specs=[
            row(128), row(128), row(16), row(16), full((_HEADS, 128)),
            full((1, 128)), full((128, 1)), full((1, 1)), full((1, 128)),
            full((1, 128)), full((128, 128)), full((1, 128)), full((1, 128)),
            full((128, _HEADS)),
        ],
        out_specs=[row(128), row(_HEADS), row(_HEADS), row(128), row(128)],
        out_shape=[
            jax.ShapeDtypeStruct((n, 128), _F32),
            jax.ShapeDtypeStruct((n, _HEADS), _F32),
            jax.ShapeDtypeStruct((n, _HEADS), _F32),
            jax.ShapeDtypeStruct((n, 128), _F32),
            jax.ShapeDtypeStruct((n, 128), _F32),
        ],
    )(p0, p1, d0, d1, jnp.asarray(_SELT4), b1, hat0T, hb0, bnw, bnb, W2,
      asf2, adf2, jnp.asarray(_SEL4))


def _tcC_body(p0_ref, p1_ref, d0_ref, d1_ref, selt_ref, b2_ref, zscale_ref,
              zsum_ref, hat1_ref, hb1_ref, bnw_ref, bnb_ref, w3_ref, as3_ref,
              ad3_ref, h_ref, asrc_ref, adst_ref):
    num = p0_ref[...] + p1_ref[...]
    den = (d0_ref[...] + d1_ref[...])[:, :_HEADS]
    denb = jnp.dot(den, selt_ref[...], preferred_element_type=_F32,
                   precision=_HIGH)
    h2 = num / (denb + 1e-16) + b2_ref[...]
    gcat = jnp.concatenate([h2, zscale_ref[...]], axis=1)
    ga = jnp.dot(_elu(gcat), hat1_ref[...], preferred_element_type=_F32,
                 precision=_HIGH) + hb1_ref[...]
    z = h2 * ga
    zs2 = zsum_ref[...] + z
    bn = (zs2 / np.float32(np.sqrt(1.0 + 1e-05))) * bnw_ref[...] + bnb_ref[...]
    h2p = _elu(bn)
    h3 = jnp.dot(h2p, w3_ref[...], preferred_element_type=_F32,
                 precision=_HIGH)
    h_ref[...] = h3
    asrc_ref[...] = jnp.dot(h3, as3_ref[...], preferred_element_type=_F32,
                            precision=_HIGH)
    adst_ref[...] = jnp.dot(h3, ad3_ref[...], preferred_element_type=_F32,
                            precision=_HIGH)


def _tcC(p0, p1, d0, d1, b2, zscale, zsum, hat1T, hb1, bnw, bnb, W3, as3T,
         ad3T):
    n = p0.shape[0]
    return pl.pallas_call(
        _tcC_body,
        out_shape=[
            jax.ShapeDtypeStruct((n, 128), _F32),
            jax.ShapeDtypeStruct((n, 1), _F32),
            jax.ShapeDtypeStruct((n, 1), _F32),
        ],
    )(p0, p1, d0, d1, jnp.asarray(_SELT4), b2, zscale, zsum, hat1T, hb1,
      bnw, bnb, W3, as3T, ad3T)


def _tcD_body(p0_ref, p1_ref, d0_ref, d1_ref, b3_ref, out_ref):
    num = p0_ref[...] + p1_ref[...]
    den = (d0_ref[...] + d1_ref[...])[:, 0:1]
    out_ref[...] = num / (den + 1e-16) + b3_ref[...]


def _tcD(p0, p1, d0, d1, b3):
    n = p0.shape[0]
    return pl.pallas_call(
        _tcD_body,
        out_shape=jax.ShapeDtypeStruct((n, 128), _F32),
    )(p0, p1, d0, d1, b3)


_MESH = plsc.VectorSubcoreMesh(core_axis_name="c", subcore_axis_name="s")


def _split(total, piece):
    out = []
    while total > 0:
        out.append(min(total, piece))
        total -= out[-1]
    return out


def _make_sc_ex(e_true, n_src, n_dst_pad, heads, nblk):
    @functools.partial(
        pl.kernel,
        mesh=_MESH,
        compiler_params=pltpu.CompilerParams(needs_layout_passes=False),
        out_type=jax.ShapeDtypeStruct((_NW * nblk * 2048,), _F32),
        scratch_types=[
            pltpu.VMEM((8, 128), jnp.int32),
            pltpu.VMEM((n_src * heads,), _F32),
            pltpu.VMEM((n_dst_pad * heads,), _F32),
            pltpu.VMEM((2048,), _F32),
        ],
    )
    def ex_kernel(pk_hbm, asrc_hbm, adst_hbm, ex_hbm,
                  pk_blk, asrc_v, adst_v, ex_v):
        c = lax.axis_index("c")
        s = lax.axis_index("s")
        wid = s * _NC + c

        pltpu.sync_copy(asrc_hbm, asrc_v)
        pltpu.sync_copy(adst_hbm, adst_v)

        zv = jnp.zeros((16,), _F32)

        def zex(i, carry):
            ex_v[pl.ds(i * 16, 16)] = zv
            return carry

        lax.fori_loop(0, 128, zex, 0)

        lanes = lax.iota(jnp.int32, 16)

        def block_body(j, carry):
            jj = j % 8
            base_e = (wid * nblk + j) * 128

            @pl.when(jj == 0)
            def _stage_pk():
                j0 = pl.multiple_of(j, 8)
                pltpu.sync_copy(pk_hbm.at[wid, pl.ds(j0, 8)], pk_blk)

            for g in range(8):
                evec = g * 16 + lanes
                pk = pk_blk[jj, pl.ds(g * 16, 16)]
                sv = lax.shift_right_logical(pk, 14)
                dv = pk & 16383
                valid = (base_e + evec) < e_true
                for h in range(heads):
                    a1 = plsc.load_gather(asrc_v, [sv * heads + h])
                    a2 = plsc.load_gather(adst_v, [dv * heads + h])
                    al = a1 + a2
                    al = jnp.maximum(al, al * 0.2)
                    ex = jnp.where(valid, jnp.exp(al), 0.0)
                    plsc.store_scatter(ex_v, [evec * 16 + h], ex)

            e0 = pl.multiple_of((wid * nblk + j) * 2048, 8)
            pltpu.sync_copy(ex_v, ex_hbm.at[pl.ds(e0, 2048)])
            return carry

        lax.fori_loop(0, nblk, block_body, 0)

    return ex_kernel


def _make_sc_agg(n_out, n_zw, n_dst_pad, heads, nblk):
    cw = 128 // heads
    zchunk = n_out // n_zw
    assert zchunk % 8 == 0 and n_zw <= _NS
    zpieces = _split(zchunk, 128)
    dchunk = n_dst_pad // _NS
    dpieces = _split(dchunk, 128)
    n_writers = min(_NS, n_dst_pad // 64)
    wchunk = n_dst_pad // n_writers
    wpieces = _split(wchunk, 128)

    @functools.partial(
        pl.kernel,
        mesh=_MESH,
        compiler_params=pltpu.CompilerParams(needs_layout_passes=False),
        out_type=[
            jax.ShapeDtypeStruct((_NC, n_out, 128), _F32),
            jax.ShapeDtypeStruct((_NC, n_dst_pad // 8, 128), _F32),
        ],
        scratch_types=[
            pltpu.VMEM((8, 128), jnp.int32),
            pltpu.VMEM((128,), jnp.int32),
            pltpu.VMEM((128,), jnp.int32),
            pltpu.VMEM((2048,), _F32),
            pltpu.VMEM((128, 128), _F32),
            pltpu.VMEM((128, 128), _F32),
            pltpu.VMEM_SHARED((n_out, 128), _F32),
            pltpu.VMEM_SHARED((n_dst_pad, 128), _F32),
            pltpu.SemaphoreType.DMA,
        ],
    )
    def agg_kernel(pk_hbm, h_hbm, ex_hbm, out_hbm, dout_hbm,
                   pk_blk, src_blk, dst_blk, ex_v, rows_v, den_v,
                   acc, dacc, sem):
        c = lax.axis_index("c")
        s = lax.axis_index("s")
        wid = s * _NC + c

        zv = jnp.zeros((16,), _F32)

        def zrow(r, carry):
            for k in range(8):
                rows_v[r, pl.ds(k * 16, 16)] = zv
                den_v[r, pl.ds(k * 16, 16)] = zv
            return carry

        lax.fori_loop(0, 128, zrow, 0)

        @pl.when(s < n_zw)
        def _zero_acc():
            zlo = s * zchunk
            off = 0
            for piece in zpieces:
                pltpu.sync_copy(rows_v.at[pl.ds(0, piece)],
                                acc.at[pl.ds(zlo + off, piece)])
                off += piece

        dlo = s * dchunk
        off = 0
        for piece in dpieces:
            pltpu.sync_copy(den_v.at[pl.ds(0, piece)],
                            dacc.at[pl.ds(dlo + off, piece)])
            off += piece
        plsc.subcore_barrier()

        def block_body(j, carry):
            jj = j % 8

            @pl.when(jj == 0)
            def _stage_pk():
                j0 = pl.multiple_of(j, 8)
                pltpu.sync_copy(pk_hbm.at[wid, pl.ds(j0, 8)], pk_blk)

            for g in range(8):
                pk = pk_blk[jj, pl.ds(g * 16, 16)]
                src_blk[pl.ds(g * 16, 16)] = lax.shift_right_logical(pk, 14)
                dst_blk[pl.ds(g * 16, 16)] = pk & 16383

            e0 = pl.multiple_of((wid * nblk + j) * 2048, 8)
            pltpu.sync_copy(ex_hbm.at[pl.ds(e0, 2048)], ex_v)
            pltpu.async_copy(h_hbm.at[src_blk], rows_v, sem).wait()

            def wrow(e, carry2):
                exv = ex_v[pl.ds(e * 16, 16)]
                den_v[e, pl.ds(0, 16)] = exv
                for h in range(heads):
                    w = jnp.full((16,), exv[h], _F32)
                    for r in range(cw // 16):
                        col = h * cw + r * 16
                        seg = rows_v[e, pl.ds(col, 16)]
                        rows_v[e, pl.ds(col, 16)] = seg * w
                return carry2

            lax.fori_loop(0, 128, wrow, 0)

            pltpu.sync_copy(rows_v, acc.at[dst_blk], add=True)
            pltpu.sync_copy(den_v, dacc.at[dst_blk], add=True)
            return carry

        lax.fori_loop(0, nblk, block_body, 0)
        plsc.subcore_barrier()

        @pl.when(s < n_zw)
        def _num_writeout():
            zlo = pl.multiple_of(s * zchunk, 8)
            pltpu.sync_copy(acc.at[pl.ds(zlo, zchunk)],
                            out_hbm.at[c, pl.ds(zlo, zchunk)])

        @pl.when(s < n_writers)
        def _den_writeout():
            wlo = s * wchunk
            off = 0
            for piece in wpieces:
                pltpu.sync_copy(dacc.at[pl.ds(wlo + off, piece)],
                                den_v.at[pl.ds(0, piece)])
                for r in range(piece):
                    rows_v[r // 8, pl.ds((r % 8) * 16, 16)] = (
                        den_v[r, pl.ds(0, 16)])
                row0 = pl.multiple_of((wlo + off) // 8, 8)
                pltpu.sync_copy(
                    rows_v.at[pl.ds(0, piece // 8)],
                    dout_hbm.at[c, pl.ds(row0, piece // 8)])
                off += piece

    return agg_kernel


_EX1 = _make_sc_ex(_E0, _N0, 4096, _HEADS, 32)
_EX2 = _make_sc_ex(_E1, _N1, 1024, _HEADS, 8)
_EX3 = _make_sc_ex(_E2, _N2, 256, 1, 8)
_AGG1 = _make_sc_agg(_N1, 10, 4096, _HEADS, 32)
_AGG2 = _make_sc_agg(1024, 16, 1024, _HEADS, 8)
_AGG3 = _make_sc_agg(256, 16, 256, 1, 8)


def _edges(edge_index, e_pad):
    src = edge_index[0].astype(jnp.int32)
    dst = edge_index[1].astype(jnp.int32)
    e = src.shape[0]
    pk = jnp.left_shift(src, 14) | dst
    return jnp.pad(pk, (0, e_pad - e)).reshape(_NW, -1, 128)


def kernel(x, edge_index0, edge_index1, edge_index2, W1, att_src1, att_dst1,
           b1, W2, att_src2, att_dst2, b2, W3, att_src3, att_dst3, b3,
           hop_att0, hop_bias0, hop_att1, hop_bias1, bn_weight, bn_bias):
    pk0 = _edges(edge_index0, 131072)
    pk1 = _edges(edge_index1, 32768)
    pk2 = _edges(edge_index2, 32768)
    asf1 = att_src1.reshape(1, 128)
    adf1 = att_dst1.reshape(1, 128)
    asf2 = att_src2.reshape(1, 128)
    adf2 = att_dst2.reshape(1, 128)
    as3T = att_src3.reshape(128, 1)
    ad3T = att_dst3.reshape(128, 1)
    b1r = b1.reshape(1, 128)
    b2r = b2.reshape(1, 128)
    b3r = b3.reshape(1, 128)
    bnw = bn_weight.reshape(1, 128)
    bnb = bn_bias.reshape(1, 128)
    hat0T = hop_att0.reshape(128, 1)
    hat1T = hop_att1.reshape(256, 1)

    h1lin, asrc1, adst1 = _tcA(x, W1, asf1, adf1)
    ex1 = _EX1(pk0, asrc1.reshape(-1),
               jnp.pad(adst1[:_N1], ((0, 96), (0, 0))).reshape(-1))
    acc1, dacc1 = _AGG1(pk0, h1lin, ex1)
    den1 = dacc1.reshape(_NC, 4096, 16)
    h2lin, asrc2, adst2, zscale, zsum = _tcB(
        acc1[0], acc1[1], den1[0, :_N1], den1[1, :_N1],
        b1r, hat0T, hop_bias0, bnw, bnb, W2, asf2, adf2)

    ex2 = _EX2(pk1, asrc2.reshape(-1),
               jnp.pad(adst2[:_N2], ((0, 24), (0, 0))).reshape(-1))
    acc2, dacc2 = _AGG2(pk1, h2lin, ex2)
    den2 = dacc2.reshape(_NC, 1024, 16)
    h3lin, asrc3, adst3 = _tcC(
        acc2[0, :_N2], acc2[1, :_N2], den2[0, :_N2], den2[1, :_N2],
        b2r, zscale[:_N2], zsum[:_N2], hat1T, hop_bias1, bnw, bnb, W3,
        as3T, ad3T)

    ex3 = _EX3(pk2, asrc3.reshape(-1),
               jnp.pad(adst3[:_N3], ((0, 6), (0, 0))).reshape(-1))
    acc3, dacc3 = _AGG3(pk2, h3lin, ex3)
    den3 = dacc3.reshape(_NC, 256, 16)
    return _tcD(acc3[0, :_N3], acc3[1, :_N3], den3[0, :_N3], den3[1, :_N3],
                b3r)


# speedup vs baseline: 13.7721x; 1.0214x over previous
import functools

import numpy as np
import jax
import jax.numpy as jnp
from jax import lax
from jax.experimental import pallas as pl
from jax.experimental.pallas import tpu as pltpu
from jax.experimental.pallas import tpu_sc as plsc

_N0, _N1, _N2, _N3 = 10000, 4000, 1000, 250
_E0, _E1, _E2 = 128000, 32000, 8000
_HID, _HEADS, _OUT = 32, 4, 128
_DW0 = float(np.log(1.0 / 1.0 + 1.0 + 1e-09))
_NC, _NS = 2, 16
_NW = _NC * _NS

_F32 = jnp.float32
_HIGH = lax.Precision.HIGHEST

_SEL4 = np.repeat(np.eye(_HEADS, dtype=np.float32), _HID, axis=0)
_SELT4 = np.ascontiguousarray(_SEL4.T)


def _tcA_body(x_ref, w_ref, asf_ref, adf_ref, sel_ref, h_ref, asrc_ref,
              adst_ref):
    h = jnp.dot(x_ref[...], w_ref[...], preferred_element_type=_F32,
                precision=_HIGH)
    h_ref[...] = h
    asrc_ref[...] = jnp.dot(h * asf_ref[...], sel_ref[...],
                            preferred_element_type=_F32, precision=_HIGH)
    adst_ref[...] = jnp.dot(h * adf_ref[...], sel_ref[...],
                            preferred_element_type=_F32, precision=_HIGH)


def _tcA(x, W1, asf, adf):
    blk = 2000
    n = x.shape[0]
    return pl.pallas_call(
        _tcA_body,
        grid=(n // blk,),
        in_specs=[
            pl.BlockSpec((blk, 128), lambda i: (i, 0)),
            pl.BlockSpec((128, 128), lambda i: (0, 0)),
            pl.BlockSpec((1, 128), lambda i: (0, 0)),
            pl.BlockSpec((1, 128), lambda i: (0, 0)),
            pl.BlockSpec((128, _HEADS), lambda i: (0, 0)),
        ],
        out_specs=[
            pl.BlockSpec((blk, 128), lambda i: (i, 0)),
            pl.BlockSpec((blk, _HEADS), lambda i: (i, 0)),
            pl.BlockSpec((blk, _HEADS), lambda i: (i, 0)),
        ],
        out_shape=[
            jax.ShapeDtypeStruct((n, 128), _F32),
            jax.ShapeDtypeStruct((n, _HEADS), _F32),
            jax.ShapeDtypeStruct((n, _HEADS), _F32),
        ],
    )(x, W1, asf, adf, jnp.asarray(_SEL4))


def _elu(x):
    return jnp.where(x > 0, x, jnp.exp(x) - 1.0)


def _tcB_body(p0_ref, p1_ref, d0_ref, d1_ref, selt_ref, b1_ref, hat0_ref,
              hb0_ref, bnw_ref, bnb_ref, w2_ref, asf_ref, adf_ref, sel_ref,
              h_ref, asrc_ref, adst_ref, zscale_ref, zsum_ref):
    num = p0_ref[...] + p1_ref[...]
    den = (d0_ref[...] + d1_ref[...])[:, :_HEADS]
    denb = jnp.dot(den, selt_ref[...], preferred_element_type=_F32,
                   precision=_HIGH)
    h1 = num / (denb + 1e-16) + b1_ref[...]
    ga = jnp.dot(_elu(h1), hat0_ref[...], preferred_element_type=_F32,
                 precision=_HIGH) + hb0_ref[...]
    z = h1 * ga
    zscale_ref[...] = z * _DW0
    zsum_ref[...] = z
    bn = (h1 / np.float32(np.sqrt(1.0 + 1e-05))) * bnw_ref[...] + bnb_ref[...]
    h1p = _elu(bn)
    h2 = jnp.dot(h1p, w2_ref[...], preferred_element_type=_F32,
                 precision=_HIGH)
    h_ref[...] = h2
    asrc_ref[...] = jnp.dot(h2 * asf_ref[...], sel_ref[...],
                            preferred_element_type=_F32, precision=_HIGH)
    adst_ref[...] = jnp.dot(h2 * adf_ref[...], sel_ref[...],
                            preferred_element_type=_F32, precision=_HIGH)


def _tcB(p0, p1, d0, d1, b1, hat0T, hb0, bnw, bnb, W2, asf2, adf2):
    blk = 2000
    n = p0.shape[0]
    full = lambda shape: pl.BlockSpec(shape, lambda i: tuple(0 for _ in shape))
    row = lambda w: pl.BlockSpec((blk, w), lambda i: (i, 0))
    return pl.pallas_call(
        _tcB_body,
        grid=(n // blk,),
        in_specs=[
            row(128), row(128), row(16), row(16), full((_HEADS, 128)),
            full((1, 128)), full((128, 1)), full((1, 1)), full((1, 128)),
            full((1, 128)), full((128, 128)), full((1, 128)), full((1, 128)),
            full((128, _HEADS)),
        ],
        out_specs=[row(128), row(_HEADS), row(_HEADS), row(128), row(128)],
        out_shape=[
            jax.ShapeDtypeStruct((n, 128), _F32),
            jax.ShapeDtypeStruct((n, _HEADS), _F32),
            jax.ShapeDtypeStruct((n, _HEADS), _F32),
            jax.ShapeDtypeStruct((n, 128), _F32),
            jax.ShapeDtypeStruct((n, 128), _F32),
        ],
    )(p0, p1, d0, d1, jnp.asarray(_SELT4), b1, hat0T, hb0, bnw, bnb, W2,
      asf2, adf2, jnp.asarray(_SEL4))


def _tcC_body(p0_ref, p1_ref, d0_ref, d1_ref, selt_ref, b2_ref, zscale_ref,
              zsum_ref, hat1_ref, hb1_ref, bnw_ref, bnb_ref, w3_ref, as3_ref,
              ad3_ref, h_ref, asrc_ref, adst_ref):
    num = p0_ref[...] + p1_ref[...]
    den = (d0_ref[...] + d1_ref[...])[:, :_HEADS]
    denb = jnp.dot(den, selt_ref[...], preferred_element_type=_F32,
                   precision=_HIGH)
    h2 = num / (denb + 1e-16) + b2_ref[...]
    gcat = jnp.concatenate([h2, zscale_ref[...]], axis=1)
    ga = jnp.dot(_elu(gcat), hat1_ref[...], preferred_element_type=_F32,
                 precision=_HIGH) + hb1_ref[...]
    z = h2 * ga
    zs2 = zsum_ref[...] + z
    bn = (zs2 / np.float32(np.sqrt(1.0 + 1e-05))) * bnw_ref[...] + bnb_ref[...]
    h2p = _elu(bn)
    h3 = jnp.dot(h2p, w3_ref[...], preferred_element_type=_F32,
                 precision=_HIGH)
    h_ref[...] = h3
    asrc_ref[...] = jnp.dot(h3, as3_ref[...], preferred_element_type=_F32,
                            precision=_HIGH)
    adst_ref[...] = jnp.dot(h3, ad3_ref[...], preferred_element_type=_F32,
                            precision=_HIGH)


def _tcC(p0, p1, d0, d1, b2, zscale, zsum, hat1T, hb1, bnw, bnb, W3, as3T,
         ad3T):
    n = p0.shape[0]
    return pl.pallas_call(
        _tcC_body,
        out_shape=[
            jax.ShapeDtypeStruct((n, 128), _F32),
            jax.ShapeDtypeStruct((n, 1), _F32),
            jax.ShapeDtypeStruct((n, 1), _F32),
        ],
    )(p0, p1, d0, d1, jnp.asarray(_SELT4), b2, zscale, zsum, hat1T, hb1,
      bnw, bnb, W3, as3T, ad3T)


def _tcD_body(p0_ref, p1_ref, d0_ref, d1_ref, b3_ref, out_ref):
    num = p0_ref[...] + p1_ref[...]
    den = (d0_ref[...] + d1_ref[...])[:, 0:1]
    out_ref[...] = num / (den + 1e-16) + b3_ref[...]


def _tcD(p0, p1, d0, d1, b3):
    n = p0.shape[0]
    return pl.pallas_call(
        _tcD_body,
        out_shape=jax.ShapeDtypeStruct((n, 128), _F32),
    )(p0, p1, d0, d1, b3)


_MESH = plsc.VectorSubcoreMesh(core_axis_name="c", subcore_axis_name="s")


def _split(total, piece):
    out = []
    while total > 0:
        out.append(min(total, piece))
        total -= out[-1]
    return out


def _make_sc_ex(e_true, n_src, n_dst_pad, heads, nblk):
    @functools.partial(
        pl.kernel,
        mesh=_MESH,
        compiler_params=pltpu.CompilerParams(needs_layout_passes=False),
        out_type=jax.ShapeDtypeStruct((_NW * nblk * 2048,), _F32),
        scratch_types=[
            pltpu.VMEM((8, 128), jnp.int32),
            pltpu.VMEM((n_src * heads,), _F32),
            pltpu.VMEM((n_dst_pad * heads,), _F32),
            pltpu.VMEM((2048,), _F32),
        ],
    )
    def ex_kernel(pk_hbm, asrc_hbm, adst_hbm, ex_hbm,
                  pk_blk, asrc_v, adst_v, ex_v):
        c = lax.axis_index("c")
        s = lax.axis_index("s")
        wid = s * _NC + c

        pltpu.sync_copy(asrc_hbm, asrc_v)
        pltpu.sync_copy(adst_hbm, adst_v)

        zv = jnp.zeros((16,), _F32)

        def zex(i, carry):
            ex_v[pl.ds(i * 16, 16)] = zv
            return carry

        lax.fori_loop(0, 128, zex, 0)

        lanes = lax.iota(jnp.int32, 16)

        def block_body(j, carry):
            jj = j % 8
            base_e = (wid * nblk + j) * 128

            @pl.when(jj == 0)
            def _stage_pk():
                j0 = pl.multiple_of(j, 8)
                pltpu.sync_copy(pk_hbm.at[wid, pl.ds(j0, 8)], pk_blk)

            for g in range(8):
                evec = g * 16 + lanes
                pk = pk_blk[jj, pl.ds(g * 16, 16)]
                sv = lax.shift_right_logical(pk, 14)
                dv = pk & 16383
                valid = (base_e + evec) < e_true
                for h in range(heads):
                    a1 = plsc.load_gather(asrc_v, [sv * heads + h])
                    a2 = plsc.load_gather(adst_v, [dv * heads + h])
                    al = a1 + a2
                    al = jnp.maximum(al, al * 0.2)
                    ex = jnp.where(valid, jnp.exp(al), 0.0)
                    plsc.store_scatter(ex_v, [evec * 16 + h], ex)

            e0 = pl.multiple_of((wid * nblk + j) * 2048, 8)
            pltpu.sync_copy(ex_v, ex_hbm.at[pl.ds(e0, 2048)])
            return carry

        lax.fori_loop(0, nblk, block_body, 0)

    return ex_kernel


def _make_sc_agg(n_out, n_zw, n_dst_pad, heads, nblk):
    cw = 128 // heads
    zchunk = n_out // n_zw
    assert zchunk % 8 == 0 and n_zw <= _NS
    zpieces = _split(zchunk, 128)
    dchunk = n_dst_pad // _NS
    dpieces = _split(dchunk, 128)
    n_writers = min(_NS, n_dst_pad // 64)
    wchunk = n_dst_pad // n_writers
    wpieces = _split(wchunk, 128)

    @functools.partial(
        pl.kernel,
        mesh=_MESH,
        compiler_params=pltpu.CompilerParams(needs_layout_passes=False),
        out_type=[
            jax.ShapeDtypeStruct((_NC, n_out, 128), _F32),
            jax.ShapeDtypeStruct((_NC, n_dst_pad // 8, 128), _F32),
        ],
        scratch_types=[
            pltpu.VMEM((8, 128), jnp.int32),
            pltpu.VMEM((2, 128), jnp.int32),
            pltpu.VMEM((2, 128), jnp.int32),
            pltpu.VMEM((2048,), _F32),
            pltpu.VMEM((2, 128, 128), _F32),
            pltpu.VMEM((128, 128), _F32),
            pltpu.VMEM_SHARED((n_out, 128), _F32),
            pltpu.VMEM_SHARED((n_dst_pad, 128), _F32),
            pltpu.SemaphoreType.DMA,
            pltpu.SemaphoreType.DMA,
        ],
    )
    def agg_kernel(pk_hbm, h_hbm, ex_hbm, out_hbm, dout_hbm,
                   pk_blk, src_blk, dst_blk, ex_v, rows2_v, den_v,
                   acc, dacc, sem0, sem1):
        c = lax.axis_index("c")
        s = lax.axis_index("s")
        wid = s * _NC + c

        zv = jnp.zeros((16,), _F32)

        def zrow(r, carry):
            for k in range(8):
                rows2_v[0, r, pl.ds(k * 16, 16)] = zv
                den_v[r, pl.ds(k * 16, 16)] = zv
            return carry

        lax.fori_loop(0, 128, zrow, 0)

        @pl.when(s < n_zw)
        def _zero_acc():
            zlo = s * zchunk
            off = 0
            for piece in zpieces:
                pltpu.sync_copy(rows2_v.at[0, pl.ds(0, piece)],
                                acc.at[pl.ds(zlo + off, piece)])
                off += piece

        dlo = s * dchunk
        off = 0
        for piece in dpieces:
            pltpu.sync_copy(den_v.at[pl.ds(0, piece)],
                            dacc.at[pl.ds(dlo + off, piece)])
            off += piece
        plsc.subcore_barrier()

        def unpack_and_prefetch(j, sem):
            jj = j % 8
            b = j & 1

            @pl.when(jj == 0)
            def _stage_pk():
                j0 = pl.multiple_of(j, 8)
                pltpu.sync_copy(pk_hbm.at[wid, pl.ds(j0, 8)], pk_blk)

            for g in range(8):
                pk = pk_blk[jj, pl.ds(g * 16, 16)]
                src_blk[b, pl.ds(g * 16, 16)] = lax.shift_right_logical(pk, 14)
                dst_blk[b, pl.ds(g * 16, 16)] = pk & 16383
            return pltpu.async_copy(h_hbm.at[src_blk.at[b]], rows2_v.at[b],
                                    sem)

        unpack_and_prefetch(0, sem0).wait()

        def block_body(j, carry):
            b = j & 1

            @pl.when(j + 1 < nblk)
            def _prefetch():
                @pl.when((j & 1) == 0)
                def _p0():
                    unpack_and_prefetch(j + 1, sem1)

                @pl.when((j & 1) == 1)
                def _p1():
                    unpack_and_prefetch(j + 1, sem0)

            e0 = pl.multiple_of((wid * nblk + j) * 2048, 8)
            pltpu.sync_copy(ex_hbm.at[pl.ds(e0, 2048)], ex_v)

            def wrow(e, carry2):
                exv = ex_v[pl.ds(e * 16, 16)]
                den_v[e, pl.ds(0, 16)] = exv
                for h in range(heads):
                    w = jnp.full((16,), exv[h], _F32)
                    for r in range(cw // 16):
                        col = h * cw + r * 16
                        seg = rows2_v[b, e, pl.ds(col, 16)]
                        rows2_v[b, e, pl.ds(col, 16)] = seg * w
                return carry2

            lax.fori_loop(0, 128, wrow, 0)

            pltpu.sync_copy(rows2_v.at[b], acc.at[dst_blk.at[b]], add=True)
            pltpu.sync_copy(den_v, dacc.at[dst_blk.at[b]], add=True)

            @pl.when(j + 1 < nblk)
            def _wait_next():
                nb = (j + 1) & 1

                @pl.when(nb == 0)
                def _w0():
                    pltpu.make_async_copy(
                        h_hbm.at[src_blk.at[0]], rows2_v.at[0], sem0).wait()

                @pl.when(nb == 1)
                def _w1():
                    pltpu.make_async_copy(
                        h_hbm.at[src_blk.at[1]], rows2_v.at[1], sem1).wait()
            return carry

        lax.fori_loop(0, nblk, block_body, 0)
        plsc.subcore_barrier()

        @pl.when(s < n_zw)
        def _num_writeout():
            zlo = pl.multiple_of(s * zchunk, 8)
            pltpu.sync_copy(acc.at[pl.ds(zlo, zchunk)],
                            out_hbm.at[c, pl.ds(zlo, zchunk)])

        @pl.when(s < n_writers)
        def _den_writeout():
            wlo = s * wchunk
            off = 0
            for piece in wpieces:
                pltpu.sync_copy(dacc.at[pl.ds(wlo + off, piece)],
                                den_v.at[pl.ds(0, piece)])
                for r in range(piece):
                    rows2_v[0, r // 8, pl.ds((r % 8) * 16, 16)] = (
                        den_v[r, pl.ds(0, 16)])
                row0 = pl.multiple_of((wlo + off) // 8, 8)
                pltpu.sync_copy(
                    rows2_v.at[0, pl.ds(0, piece // 8)],
                    dout_hbm.at[c, pl.ds(row0, piece // 8)])
                off += piece

    return agg_kernel


_EX1 = _make_sc_ex(_E0, _N0, 4096, _HEADS, 32)
_EX2 = _make_sc_ex(_E1, _N1, 1024, _HEADS, 8)
_EX3 = _make_sc_ex(_E2, _N2, 256, 1, 8)
_AGG1 = _make_sc_agg(_N1, 10, 4096, _HEADS, 32)
_AGG2 = _make_sc_agg(1024, 16, 1024, _HEADS, 8)
_AGG3 = _make_sc_agg(256, 16, 256, 1, 8)


def _edges(edge_index, e_pad):
    src = edge_index[0].astype(jnp.int32)
    dst = edge_index[1].astype(jnp.int32)
    e = src.shape[0]
    pk = jnp.left_shift(src, 14) | dst
    return jnp.pad(pk, (0, e_pad - e)).reshape(_NW, -1, 128)


def kernel(x, edge_index0, edge_index1, edge_index2, W1, att_src1, att_dst1,
           b1, W2, att_src2, att_dst2, b2, W3, att_src3, att_dst3, b3,
           hop_att0, hop_bias0, hop_att1, hop_bias1, bn_weight, bn_bias):
    pk0 = _edges(edge_index0, 131072)
    pk1 = _edges(edge_index1, 32768)
    pk2 = _edges(edge_index2, 32768)
    asf1 = att_src1.reshape(1, 128)
    adf1 = att_dst1.reshape(1, 128)
    asf2 = att_src2.reshape(1, 128)
    adf2 = att_dst2.reshape(1, 128)
    as3T = att_src3.reshape(128, 1)
    ad3T = att_dst3.reshape(128, 1)
    b1r = b1.reshape(1, 128)
    b2r = b2.reshape(1, 128)
    b3r = b3.reshape(1, 128)
    bnw = bn_weight.reshape(1, 128)
    bnb = bn_bias.reshape(1, 128)
    hat0T = hop_att0.reshape(128, 1)
    hat1T = hop_att1.reshape(256, 1)

    h1lin, asrc1, adst1 = _tcA(x, W1, asf1, adf1)
    ex1 = _EX1(pk0, asrc1.reshape(-1),
               jnp.pad(adst1[:_N1], ((0, 96), (0, 0))).reshape(-1))
    acc1, dacc1 = _AGG1(pk0, h1lin, ex1)
    den1 = dacc1.reshape(_NC, 4096, 16)
    h2lin, asrc2, adst2, zscale, zsum = _tcB(
        acc1[0], acc1[1], den1[0, :_N1], den1[1, :_N1],
        b1r, hat0T, hop_bias0, bnw, bnb, W2, asf2, adf2)

    ex2 = _EX2(pk1, asrc2.reshape(-1),
               jnp.pad(adst2[:_N2], ((0, 24), (0, 0))).reshape(-1))
    acc2, dacc2 = _AGG2(pk1, h2lin, ex2)
    den2 = dacc2.reshape(_NC, 1024, 16)
    h3lin, asrc3, adst3 = _tcC(
        acc2[0, :_N2], acc2[1, :_N2], den2[0, :_N2], den2[1, :_N2],
        b2r, zscale[:_N2], zsum[:_N2], hat1T, hop_bias1, bnw, bnb, W3,
        as3T, ad3T)

    ex3 = _EX3(pk2, asrc3.reshape(-1),
               jnp.pad(adst3[:_N3], ((0, 6), (0, 0))).reshape(-1))
    acc3, dacc3 = _AGG3(pk2, h3lin, ex3)
    den3 = dacc3.reshape(_NC, 256, 16)
    return _tcD(acc3[0, :_N3], acc3[1, :_N3], den3[0, :_N3], den3[1, :_N3],
                b3r)
